# Initial kernel scaffold; baseline (speedup 1.0000x reference)
import jax
import jax.numpy as jnp
from jax.experimental import pallas as pl


def kernel(input_matrix, adj, W1_self, W1_neigh, b1, W2_self, W2_neigh, b2):
    raise NotImplementedError("write your pallas kernel here")


# speedup vs baseline: 4.3254x; 4.3254x over previous
import jax
import jax.numpy as jnp
from jax import lax
from jax.experimental import pallas as pl
from jax.experimental.pallas import tpu as pltpu
from jax.experimental.pallas import tpu_sc as plsc

N = 10000
E = 160000
D_IN = 256
D_HID = 256
NCLS = 40
NCP = 48

NC, NS = 2, 16
G = 128
EPAD = NC * NS * 40 * G
NGRP = EPAD // G
FEAT_GPT = NGRP // NS
DEG_GPT = NGRP // (NC * NS)
ACC_ROWS = 10240
ZCHUNK = ACC_ROWS // NS
OCHUNK = 624
OLAST = N - (NS - 1) * OCHUNK
CH = 8

def _sc1_body(p1a, p1b, srcg, dstg, z128, z16, ones_in,
              m1, dg,
              acc, dacc, sidx, didx, didxd, rows, ones, sem):
    c = lax.axis_index("c")
    s = lax.axis_index("s")

    pltpu.sync_copy(z128.at[pl.ds(s * ZCHUNK, ZCHUNK)],
                    acc.at[pl.ds(s * ZCHUNK, ZCHUNK)])
    pltpu.sync_copy(z16.at[pl.ds(s * ZCHUNK, ZCHUNK)],
                    dacc.at[pl.ds(s * ZCHUNK, ZCHUNK)])

    pltpu.sync_copy(ones_in, ones)
    plsc.subcore_barrier()

    wid = c * NS + s

    def _degchunk(ch, carry):
        pltpu.sync_copy(dstg.at[pl.ds(wid * DEG_GPT + ch * CH, CH)], didxd)

        def _deg(j, carry2):
            pltpu.sync_copy(ones, dacc.at[didxd.at[j]], add=True)
            return carry2

        lax.fori_loop(0, CH, _deg, 0)
        return carry

    lax.fori_loop(0, DEG_GPT // CH, _degchunk, 0)

    fbase = s * FEAT_GPT

    def _run(table):
        def _chunk(ch, carry):
            pltpu.sync_copy(srcg.at[pl.ds(fbase + ch * CH, CH)], sidx)
            pltpu.sync_copy(dstg.at[pl.ds(fbase + ch * CH, CH)], didx)

            def _grp(g, carry2):
                pltpu.async_copy(table.at[sidx.at[g]], rows, sem).wait()
                pltpu.sync_copy(rows, acc.at[didx.at[g]], add=True)
                return carry2

            lax.fori_loop(0, CH, _grp, 0)
            return carry

        lax.fori_loop(0, FEAT_GPT // CH, _chunk, 0)

    @pl.when(c == 0)
    def _():
        _run(p1a)

    @pl.when(c == 1)
    def _():
        _run(p1b)

    plsc.subcore_barrier()

    @pl.when(s < NS - 1)
    def _():
        pltpu.sync_copy(acc.at[pl.ds(s * OCHUNK, OCHUNK)],
                        m1.at[pl.ds(c * N + s * OCHUNK, OCHUNK)])
        pltpu.sync_copy(dacc.at[pl.ds(s * OCHUNK, OCHUNK)],
                        dg.at[pl.ds(c * N + s * OCHUNK, OCHUNK)])

    @pl.when(s == NS - 1)
    def _():
        pltpu.sync_copy(acc.at[pl.ds((NS - 1) * OCHUNK, OLAST)],
                        m1.at[pl.ds(c * N + (NS - 1) * OCHUNK, OLAST)])
        pltpu.sync_copy(dacc.at[pl.ds((NS - 1) * OCHUNK, OLAST)],
                        dg.at[pl.ds(c * N + (NS - 1) * OCHUNK, OLAST)])


_sc_cache = {}


def _sc1(*args):
    k = _sc_cache.get("sc1")
    if k is None:
        mesh = plsc.VectorSubcoreMesh(core_axis_name="c", subcore_axis_name="s")
        k = _sc_cache["sc1"] = pl.kernel(
            _sc1_body,
            out_type=[jax.ShapeDtypeStruct((2 * N, 128), jnp.float32),
                      jax.ShapeDtypeStruct((2 * N, 16), jnp.float32)],
            mesh=mesh,
            scratch_types=[
                pltpu.VMEM_SHARED((ACC_ROWS, 128), jnp.float32),
                pltpu.VMEM_SHARED((ACC_ROWS, 16), jnp.float32),
                pltpu.VMEM((CH, G), jnp.int32),
                pltpu.VMEM((CH, G), jnp.int32),
                pltpu.VMEM((CH, G), jnp.int32),
                pltpu.VMEM((G, 128), jnp.float32),
                pltpu.VMEM((G, 16), jnp.float32),
                pltpu.SemaphoreType.DMA,
            ],
            compiler_params=pltpu.CompilerParams(use_tc_tiling_on_sc=False),
        )
    return k(*args)


def _sc2_body(p2, srcg, dstg, z48,
              m2,
              acc, sidx, didx, rows, sem):
    c = lax.axis_index("c")
    s = lax.axis_index("s")

    pltpu.sync_copy(z48.at[pl.ds(s * ZCHUNK, ZCHUNK)],
                    acc.at[pl.ds(s * ZCHUNK, ZCHUNK)])
    plsc.subcore_barrier()

    wid = c * NS + s
    base = wid * DEG_GPT
    pltpu.sync_copy(srcg.at[pl.ds(base, DEG_GPT)], sidx)
    pltpu.sync_copy(dstg.at[pl.ds(base, DEG_GPT)], didx)

    def _grp(g, carry):
        pltpu.async_copy(p2.at[sidx.at[g]], rows, sem).wait()
        pltpu.sync_copy(rows, acc.at[didx.at[g]], add=True)
        return carry

    lax.fori_loop(0, DEG_GPT, _grp, 0)
    plsc.subcore_barrier()

    @pl.when(s < NS - 1)
    def _():
        pltpu.sync_copy(acc.at[pl.ds(s * OCHUNK, OCHUNK)],
                        m2.at[pl.ds(c * N + s * OCHUNK, OCHUNK)])

    @pl.when(s == NS - 1)
    def _():
        pltpu.sync_copy(acc.at[pl.ds((NS - 1) * OCHUNK, OLAST)],
                        m2.at[pl.ds(c * N + (NS - 1) * OCHUNK, OLAST)])


def _sc2(*args):
    k = _sc_cache.get("sc2")
    if k is None:
        mesh = plsc.VectorSubcoreMesh(core_axis_name="c", subcore_axis_name="s")
        k = _sc_cache["sc2"] = pl.kernel(
            _sc2_body,
            out_type=jax.ShapeDtypeStruct((2 * N, NCP), jnp.float32),
            mesh=mesh,
            scratch_types=[
                pltpu.VMEM_SHARED((ACC_ROWS, NCP), jnp.float32),
                pltpu.VMEM((DEG_GPT, G), jnp.int32),
                pltpu.VMEM((DEG_GPT, G), jnp.int32),
                pltpu.VMEM((G, NCP), jnp.float32),
                pltpu.SemaphoreType.DMA,
            ],
            compiler_params=pltpu.CompilerParams(use_tc_tiling_on_sc=False),
        )
    return k(*args)


_MB = 1000


def _tc1_body(x, w, b, s1, p1a, p1b):
    z = jnp.dot(x[:], w[:], preferred_element_type=jnp.float32) + b[:]
    s1[:] = z[:, :D_HID]
    p1a[:] = z[:, D_HID:D_HID + 128]
    p1b[:] = z[:, D_HID + 128:]


def _tc2_body(s1, m1, dg, w, b, s2, p2):
    dgb = dg[:]
    deg = dgb[0] + dgb[1]
    rdeg = 1.0 / jnp.maximum(deg[:, 0:1], 1.0)
    m1b = m1[:]
    mean1 = jnp.concatenate([m1b[0], m1b[1]], axis=1) * rdeg
    h = jnp.maximum(s1[:] + mean1, 0.0)
    z = jnp.dot(h, w[:], preferred_element_type=jnp.float32) + b[:]
    s2[:] = z[:, :NCP]
    p2[:] = z[:, NCP:]


def _tc3_body(s2, m2, dg, out):
    dgb = dg[:]
    deg = dgb[0] + dgb[1]
    rdeg = 1.0 / jnp.maximum(deg[:, 0:1], 1.0)
    m2b = m2[:]
    out[:] = s2[:] + (m2b[0] + m2b[1]) * rdeg


def kernel(input_matrix, adj, W1_self, W1_neigh, b1, W2_self, W2_neigh, b2):
    f32 = jnp.float32
    x = input_matrix.astype(f32)

    src = adj[0].astype(jnp.int32)
    dst = adj[1].astype(jnp.int32)
    pad = EPAD - E
    srcg = jnp.concatenate([src, jnp.zeros((pad,), jnp.int32)]).reshape(NGRP, G)
    dstg = jnp.concatenate([dst, jnp.full((pad,), N, jnp.int32)]).reshape(NGRP, G)

    w1 = jnp.concatenate([W1_self, W1_neigh], axis=1)
    bc1 = jnp.concatenate([b1, jnp.zeros((D_HID,), f32)]).reshape(1, 2 * D_HID)

    zpad = jnp.zeros((D_HID, NCP - NCLS), f32)
    w2 = jnp.concatenate([W2_self, zpad, W2_neigh, zpad], axis=1)
    bc2 = jnp.concatenate([b2, jnp.zeros((2 * NCP - NCLS,), f32)]).reshape(1, 2 * NCP)

    z128 = jnp.zeros((ACC_ROWS, 128), f32)
    z16 = jnp.zeros((ACC_ROWS, 16), f32)
    z48 = jnp.zeros((ACC_ROWS, NCP), f32)

    s1, p1a, p1b = pl.pallas_call(
        _tc1_body,
        grid=(N // _MB,),
        in_specs=[pl.BlockSpec((_MB, D_IN), lambda i: (i, 0)),
                  pl.BlockSpec((D_IN, 2 * D_HID), lambda i: (0, 0)),
                  pl.BlockSpec((1, 2 * D_HID), lambda i: (0, 0))],
        out_specs=[pl.BlockSpec((_MB, D_HID), lambda i: (i, 0)),
                   pl.BlockSpec((_MB, 128), lambda i: (i, 0)),
                   pl.BlockSpec((_MB, 128), lambda i: (i, 0))],
        out_shape=[jax.ShapeDtypeStruct((N, D_HID), f32),
                   jax.ShapeDtypeStruct((N, 128), f32),
                   jax.ShapeDtypeStruct((N, 128), f32)],
    )(x, w1, bc1)

    ones_in = jnp.ones((G, 16), f32)
    m1, dg = _sc1(p1a, p1b, srcg, dstg, z128, z16, ones_in)
    m1 = m1.reshape(2, N, 128)
    dg = dg.reshape(2, N, 16)

    s2, p2 = pl.pallas_call(
        _tc2_body,
        grid=(N // _MB,),
        in_specs=[pl.BlockSpec((_MB, D_HID), lambda i: (i, 0)),
                  pl.BlockSpec((2, _MB, 128), lambda i: (0, i, 0)),
                  pl.BlockSpec((2, _MB, 16), lambda i: (0, i, 0)),
                  pl.BlockSpec((D_HID, 2 * NCP), lambda i: (0, 0)),
                  pl.BlockSpec((1, 2 * NCP), lambda i: (0, 0))],
        out_specs=[pl.BlockSpec((_MB, NCP), lambda i: (i, 0)),
                   pl.BlockSpec((_MB, NCP), lambda i: (i, 0))],
        out_shape=[jax.ShapeDtypeStruct((N, NCP), f32),
                   jax.ShapeDtypeStruct((N, NCP), f32)],
    )(s1, m1, dg, w2, bc2)

    m2 = _sc2(p2, srcg, dstg, z48).reshape(2, N, NCP)

    out = pl.pallas_call(
        _tc3_body,
        grid=(N // _MB,),
        in_specs=[pl.BlockSpec((_MB, NCP), lambda i: (i, 0)),
                  pl.BlockSpec((2, _MB, NCP), lambda i: (0, i, 0)),
                  pl.BlockSpec((2, _MB, 16), lambda i: (0, i, 0))],
        out_specs=pl.BlockSpec((_MB, NCP), lambda i: (i, 0)),
        out_shape=jax.ShapeDtypeStruct((N, NCP), f32),
    )(s2, m2, dg)

    return out[:, :NCLS]


# speedup vs baseline: 4.9134x; 1.1359x over previous
import jax
import jax.numpy as jnp
from jax import lax
from jax.experimental import pallas as pl
from jax.experimental.pallas import tpu as pltpu
from jax.experimental.pallas import tpu_sc as plsc

N = 10000
E = 160000
D_IN = 256
D_HID = 256
NCLS = 40
NCP = 48

NC, NS = 2, 16
G = 128
EPAD = NC * NS * 40 * G
NGRP = EPAD // G
FEAT_GPT = NGRP // NS
DEG_GPT = NGRP // (NC * NS)
ACC_ROWS = 10240
ZCHUNK = ACC_ROWS // NS
OCHUNK = 624
OLAST = N - (NS - 1) * OCHUNK
CH = 8

def _sc1_body(p1a, p1b, srcg, dstg, z128, z16, ones_in,
              m1, dg,
              acc, dacc, sidx, didx, rows0, rows1, ones,
              gsem0, gsem1, ssem0, ssem1):
    c = lax.axis_index("c")
    s = lax.axis_index("s")

    pltpu.sync_copy(z128.at[pl.ds(s * ZCHUNK, ZCHUNK)],
                    acc.at[pl.ds(s * ZCHUNK, ZCHUNK)])
    pltpu.sync_copy(z16.at[pl.ds(s * ZCHUNK, ZCHUNK)],
                    dacc.at[pl.ds(s * ZCHUNK, ZCHUNK)])

    pltpu.sync_copy(ones_in, ones)
    plsc.subcore_barrier()

    wid = c * NS + s

    def _degchunk(ch, carry):
        pltpu.sync_copy(dstg.at[pl.ds(wid * DEG_GPT + ch * CH, CH)], didx)
        descs = [pltpu.async_copy(ones, dacc.at[didx.at[j]], gsem0, add=True)
                 for j in range(CH)]
        for d in descs:
            d.wait()
        return carry

    lax.fori_loop(0, DEG_GPT // CH, _degchunk, 0)

    fbase = s * FEAT_GPT

    def _run(table):
        rbufs = (rows0, rows1)
        gsems = (gsem0, gsem1)
        ssems = (ssem0, ssem1)

        def _chunk(ch, carry):
            pltpu.sync_copy(srcg.at[pl.ds(fbase + ch * CH, CH)], sidx)
            pltpu.sync_copy(dstg.at[pl.ds(fbase + ch * CH, CH)], didx)
            gd = [None, None]
            sd = [None, None]
            gd[0] = pltpu.async_copy(table.at[sidx.at[0]], rbufs[0], gsems[0])
            for j in range(CH):
                b = j & 1
                nb = (j + 1) & 1
                if j + 1 < CH:
                    if sd[nb] is not None:
                        sd[nb].wait()
                    gd[nb] = pltpu.async_copy(table.at[sidx.at[j + 1]],
                                              rbufs[nb], gsems[nb])
                gd[b].wait()
                sd[b] = pltpu.async_copy(rbufs[b], acc.at[didx.at[j]],
                                         ssems[b], add=True)
            sd[0].wait()
            sd[1].wait()
            return carry

        lax.fori_loop(0, FEAT_GPT // CH, _chunk, 0)

    @pl.when(c == 0)
    def _():
        _run(p1a)

    @pl.when(c == 1)
    def _():
        _run(p1b)

    plsc.subcore_barrier()

    @pl.when(s < NS - 1)
    def _():
        pltpu.sync_copy(acc.at[pl.ds(s * OCHUNK, OCHUNK)],
                        m1.at[pl.ds(c * N + s * OCHUNK, OCHUNK)])
        pltpu.sync_copy(dacc.at[pl.ds(s * OCHUNK, OCHUNK)],
                        dg.at[pl.ds(c * N + s * OCHUNK, OCHUNK)])

    @pl.when(s == NS - 1)
    def _():
        pltpu.sync_copy(acc.at[pl.ds((NS - 1) * OCHUNK, OLAST)],
                        m1.at[pl.ds(c * N + (NS - 1) * OCHUNK, OLAST)])
        pltpu.sync_copy(dacc.at[pl.ds((NS - 1) * OCHUNK, OLAST)],
                        dg.at[pl.ds(c * N + (NS - 1) * OCHUNK, OLAST)])


_sc_cache = {}


def _sc1(*args):
    k = _sc_cache.get("sc1")
    if k is None:
        mesh = plsc.VectorSubcoreMesh(core_axis_name="c", subcore_axis_name="s")
        k = _sc_cache["sc1"] = pl.kernel(
            _sc1_body,
            out_type=[jax.ShapeDtypeStruct((2 * N, 128), jnp.float32),
                      jax.ShapeDtypeStruct((2 * N, 16), jnp.float32)],
            mesh=mesh,
            scratch_types=[
                pltpu.VMEM_SHARED((ACC_ROWS, 128), jnp.float32),
                pltpu.VMEM_SHARED((ACC_ROWS, 16), jnp.float32),
                pltpu.VMEM((CH, G), jnp.int32),
                pltpu.VMEM((CH, G), jnp.int32),
                pltpu.VMEM((G, 128), jnp.float32),
                pltpu.VMEM((G, 128), jnp.float32),
                pltpu.VMEM((G, 16), jnp.float32),
                pltpu.SemaphoreType.DMA,
                pltpu.SemaphoreType.DMA,
                pltpu.SemaphoreType.DMA,
                pltpu.SemaphoreType.DMA,
            ],
            compiler_params=pltpu.CompilerParams(use_tc_tiling_on_sc=False),
        )
    return k(*args)


def _sc2_body(p2, srcg, dstg, z48,
              m2,
              acc, sidx, didx, rows0, rows1,
              gsem0, gsem1, ssem0, ssem1):
    c = lax.axis_index("c")
    s = lax.axis_index("s")

    pltpu.sync_copy(z48.at[pl.ds(s * ZCHUNK, ZCHUNK)],
                    acc.at[pl.ds(s * ZCHUNK, ZCHUNK)])
    plsc.subcore_barrier()

    wid = c * NS + s
    base = wid * DEG_GPT
    rbufs = (rows0, rows1)
    gsems = (gsem0, gsem1)
    ssems = (ssem0, ssem1)

    def _chunk(ch, carry):
        pltpu.sync_copy(srcg.at[pl.ds(base + ch * CH, CH)], sidx)
        pltpu.sync_copy(dstg.at[pl.ds(base + ch * CH, CH)], didx)
        gd = [None, None]
        sd = [None, None]
        gd[0] = pltpu.async_copy(p2.at[sidx.at[0]], rbufs[0], gsems[0])
        for j in range(CH):
            b = j & 1
            nb = (j + 1) & 1
            if j + 1 < CH:
                if sd[nb] is not None:
                    sd[nb].wait()
                gd[nb] = pltpu.async_copy(p2.at[sidx.at[j + 1]],
                                          rbufs[nb], gsems[nb])
            gd[b].wait()
            sd[b] = pltpu.async_copy(rbufs[b], acc.at[didx.at[j]],
                                     ssems[b], add=True)
        sd[0].wait()
        sd[1].wait()
        return carry

    lax.fori_loop(0, DEG_GPT // CH, _chunk, 0)
    plsc.subcore_barrier()

    @pl.when(s < NS - 1)
    def _():
        pltpu.sync_copy(acc.at[pl.ds(s * OCHUNK, OCHUNK)],
                        m2.at[pl.ds(c * N + s * OCHUNK, OCHUNK)])

    @pl.when(s == NS - 1)
    def _():
        pltpu.sync_copy(acc.at[pl.ds((NS - 1) * OCHUNK, OLAST)],
                        m2.at[pl.ds(c * N + (NS - 1) * OCHUNK, OLAST)])


def _sc2(*args):
    k = _sc_cache.get("sc2")
    if k is None:
        mesh = plsc.VectorSubcoreMesh(core_axis_name="c", subcore_axis_name="s")
        k = _sc_cache["sc2"] = pl.kernel(
            _sc2_body,
            out_type=jax.ShapeDtypeStruct((2 * N, NCP), jnp.float32),
            mesh=mesh,
            scratch_types=[
                pltpu.VMEM_SHARED((ACC_ROWS, NCP), jnp.float32),
                pltpu.VMEM((CH, G), jnp.int32),
                pltpu.VMEM((CH, G), jnp.int32),
                pltpu.VMEM((G, NCP), jnp.float32),
                pltpu.VMEM((G, NCP), jnp.float32),
                pltpu.SemaphoreType.DMA,
                pltpu.SemaphoreType.DMA,
                pltpu.SemaphoreType.DMA,
                pltpu.SemaphoreType.DMA,
            ],
            compiler_params=pltpu.CompilerParams(use_tc_tiling_on_sc=False),
        )
    return k(*args)


_MB = 1000


def _tc1_body(x, w, b, s1, p1a, p1b):
    z = jnp.dot(x[:], w[:], preferred_element_type=jnp.float32) + b[:]
    s1[:] = z[:, :D_HID]
    p1a[:] = z[:, D_HID:D_HID + 128]
    p1b[:] = z[:, D_HID + 128:]


def _tc2_body(s1, m1, dg, w, b, s2, p2):
    dgb = dg[:]
    deg = dgb[0] + dgb[1]
    rdeg = 1.0 / jnp.maximum(deg[:, 0:1], 1.0)
    m1b = m1[:]
    mean1 = jnp.concatenate([m1b[0], m1b[1]], axis=1) * rdeg
    h = jnp.maximum(s1[:] + mean1, 0.0)
    z = jnp.dot(h, w[:], preferred_element_type=jnp.float32) + b[:]
    s2[:] = z[:, :NCP]
    p2[:] = z[:, NCP:]


def _tc3_body(s2, m2, dg, out):
    dgb = dg[:]
    deg = dgb[0] + dgb[1]
    rdeg = 1.0 / jnp.maximum(deg[:, 0:1], 1.0)
    m2b = m2[:]
    out[:] = s2[:] + (m2b[0] + m2b[1]) * rdeg


def kernel(input_matrix, adj, W1_self, W1_neigh, b1, W2_self, W2_neigh, b2):
    f32 = jnp.float32
    x = input_matrix.astype(f32)

    src = adj[0].astype(jnp.int32)
    dst = adj[1].astype(jnp.int32)
    pad = EPAD - E
    srcg = jnp.concatenate([src, jnp.zeros((pad,), jnp.int32)]).reshape(NGRP, G)
    dstg = jnp.concatenate([dst, jnp.full((pad,), N, jnp.int32)]).reshape(NGRP, G)

    w1 = jnp.concatenate([W1_self, W1_neigh], axis=1)
    bc1 = jnp.concatenate([b1, jnp.zeros((D_HID,), f32)]).reshape(1, 2 * D_HID)

    zpad = jnp.zeros((D_HID, NCP - NCLS), f32)
    w2 = jnp.concatenate([W2_self, zpad, W2_neigh, zpad], axis=1)
    bc2 = jnp.concatenate([b2, jnp.zeros((2 * NCP - NCLS,), f32)]).reshape(1, 2 * NCP)

    z128 = jnp.zeros((ACC_ROWS, 128), f32)
    z16 = jnp.zeros((ACC_ROWS, 16), f32)
    z48 = jnp.zeros((ACC_ROWS, NCP), f32)

    s1, p1a, p1b = pl.pallas_call(
        _tc1_body,
        grid=(N // _MB,),
        in_specs=[pl.BlockSpec((_MB, D_IN), lambda i: (i, 0)),
                  pl.BlockSpec((D_IN, 2 * D_HID), lambda i: (0, 0)),
                  pl.BlockSpec((1, 2 * D_HID), lambda i: (0, 0))],
        out_specs=[pl.BlockSpec((_MB, D_HID), lambda i: (i, 0)),
                   pl.BlockSpec((_MB, 128), lambda i: (i, 0)),
                   pl.BlockSpec((_MB, 128), lambda i: (i, 0))],
        out_shape=[jax.ShapeDtypeStruct((N, D_HID), f32),
                   jax.ShapeDtypeStruct((N, 128), f32),
                   jax.ShapeDtypeStruct((N, 128), f32)],
    )(x, w1, bc1)

    ones_in = jnp.ones((G, 16), f32)
    m1, dg = _sc1(p1a, p1b, srcg, dstg, z128, z16, ones_in)
    m1 = m1.reshape(2, N, 128)
    dg = dg.reshape(2, N, 16)

    s2, p2 = pl.pallas_call(
        _tc2_body,
        grid=(N // _MB,),
        in_specs=[pl.BlockSpec((_MB, D_HID), lambda i: (i, 0)),
                  pl.BlockSpec((2, _MB, 128), lambda i: (0, i, 0)),
                  pl.BlockSpec((2, _MB, 16), lambda i: (0, i, 0)),
                  pl.BlockSpec((D_HID, 2 * NCP), lambda i: (0, 0)),
                  pl.BlockSpec((1, 2 * NCP), lambda i: (0, 0))],
        out_specs=[pl.BlockSpec((_MB, NCP), lambda i: (i, 0)),
                   pl.BlockSpec((_MB, NCP), lambda i: (i, 0))],
        out_shape=[jax.ShapeDtypeStruct((N, NCP), f32),
                   jax.ShapeDtypeStruct((N, NCP), f32)],
    )(s1, m1, dg, w2, bc2)

    m2 = _sc2(p2, srcg, dstg, z48).reshape(2, N, NCP)

    out = pl.pallas_call(
        _tc3_body,
        grid=(N // _MB,),
        in_specs=[pl.BlockSpec((_MB, NCP), lambda i: (i, 0)),
                  pl.BlockSpec((2, _MB, NCP), lambda i: (0, i, 0)),
                  pl.BlockSpec((2, _MB, 16), lambda i: (0, i, 0))],
        out_specs=pl.BlockSpec((_MB, NCP), lambda i: (i, 0)),
        out_shape=jax.ShapeDtypeStruct((N, NCP), f32),
    )(s2, m2, dg)

    return out[:, :NCLS]
